# baseline (speedup 1.0000x reference)
import functools

import jax
import jax.numpy as jnp
from jax import lax
from jax.experimental import pallas as pl
from jax.experimental.pallas import tpu as pltpu
from jax.experimental.pallas import tpu_sc as plsc

B = 16384
D = 64
NUM_CORES = 2
NUM_SUBCORES = 16
NW = NUM_CORES * NUM_SUBCORES
BPW = B // NW
L = 16
TG = 16


@functools.partial(
    pl.kernel,
    out_type=(
        jax.ShapeDtypeStruct((B,), jnp.float32),
        jax.ShapeDtypeStruct((B,), jnp.float32),
    ),
    mesh=plsc.VectorSubcoreMesh(core_axis_name="c", subcore_axis_name="s"),
    scratch_types=[
        pltpu.VMEM((BPW,), jnp.int32),
        pltpu.VMEM((BPW,), jnp.int32),
        pltpu.VMEM((BPW,), jnp.int32),
        pltpu.VMEM((L, TG, D), jnp.float32),
        pltpu.VMEM((L, TG, D), jnp.float32),
        pltpu.VMEM((L, TG, D), jnp.float32),
        pltpu.VMEM((BPW,), jnp.float32),
        pltpu.VMEM((BPW,), jnp.float32),
        pltpu.SemaphoreType.DMA,
    ],
    compiler_params=pltpu.CompilerParams(needs_layout_passes=False),
)
def _mfbpr(user_h, item_i_h, item_j_h, eu_h, ei_h, oi_h, oj_h,
           idx_u, idx_i, idx_j, tb_u, tb_i, tb_j, oi_v, oj_v, sem):
    wid = lax.axis_index("s") * NUM_CORES + lax.axis_index("c")
    base = wid * BPW
    pltpu.sync_copy(user_h.at[pl.ds(base, BPW)], idx_u)
    pltpu.sync_copy(item_i_h.at[pl.ds(base, BPW)], idx_i)
    pltpu.sync_copy(item_j_h.at[pl.ds(base, BPW)], idx_j)

    lanes = lax.iota(jnp.int32, L)

    def chunk_body(c, carry):
        rbase = c * L
        iu = idx_u[pl.ds(rbase, L)]
        ii = idx_i[pl.ds(rbase, L)]
        ij = idx_j[pl.ds(rbase, L)]
        copies = []
        for t in range(L):
            gu = (iu[t] >> 4) * TG
            gi = (ii[t] >> 4) * TG
            gj = (ij[t] >> 4) * TG
            copies.append(
                pltpu.async_copy(eu_h.at[pl.ds(gu, TG)], tb_u.at[t], sem))
            copies.append(
                pltpu.async_copy(ei_h.at[pl.ds(gi, TG)], tb_i.at[t], sem))
            copies.append(
                pltpu.async_copy(ei_h.at[pl.ds(gj, TG)], tb_j.at[t], sem))
        for cp in copies:
            cp.wait()
        ru = iu & 15
        ri = ii & 15
        rj = ij & 15
        acc_i = jnp.zeros((L,), jnp.float32)
        acc_j = jnp.zeros((L,), jnp.float32)
        for k in range(D):
            col = jnp.full((L,), k, dtype=jnp.int32)
            u = plsc.load_gather(tb_u, [lanes, ru, col])
            vi = plsc.load_gather(tb_i, [lanes, ri, col])
            vj = plsc.load_gather(tb_j, [lanes, rj, col])
            acc_i = acc_i + u * vi
            acc_j = acc_j + u * vj
        oi_v[pl.ds(rbase, L)] = acc_i
        oj_v[pl.ds(rbase, L)] = acc_j
        return carry

    lax.fori_loop(0, BPW // L, chunk_body, 0)

    pltpu.sync_copy(oi_v, oi_h.at[pl.ds(base, BPW)])
    pltpu.sync_copy(oj_v, oj_h.at[pl.ds(base, BPW)])


def kernel(user, item_i, item_j, embed_user, embed_item):
    return _mfbpr(user.astype(jnp.int32), item_i.astype(jnp.int32),
                  item_j.astype(jnp.int32), embed_user, embed_item)


# speedup vs baseline: 1.0752x; 1.0752x over previous
import functools

import jax
import jax.numpy as jnp
from jax import lax
from jax.experimental import pallas as pl
from jax.experimental.pallas import tpu as pltpu
from jax.experimental.pallas import tpu_sc as plsc

B = 16384
D = 64
NUM_CORES = 2
NUM_SUBCORES = 16
NW = NUM_CORES * NUM_SUBCORES
BPW = B // NW
L = 16
TG = 8


@functools.partial(
    pl.kernel,
    out_type=(
        jax.ShapeDtypeStruct((B,), jnp.float32),
        jax.ShapeDtypeStruct((B,), jnp.float32),
    ),
    mesh=plsc.VectorSubcoreMesh(core_axis_name="c", subcore_axis_name="s"),
    scratch_types=[
        pltpu.VMEM((BPW,), jnp.int32),
        pltpu.VMEM((BPW,), jnp.int32),
        pltpu.VMEM((BPW,), jnp.int32),
        pltpu.VMEM((L, TG, D), jnp.float32),
        pltpu.VMEM((L, TG, D), jnp.float32),
        pltpu.VMEM((L, TG, D), jnp.float32),
        pltpu.VMEM((BPW,), jnp.float32),
        pltpu.VMEM((BPW,), jnp.float32),
        pltpu.SemaphoreType.DMA,
    ],
    compiler_params=pltpu.CompilerParams(needs_layout_passes=False),
)
def _mfbpr(user_h, item_i_h, item_j_h, eu_h, ei_h, oi_h, oj_h,
           idx_u, idx_i, idx_j, tb_u, tb_i, tb_j, oi_v, oj_v, sem):
    wid = lax.axis_index("s") * NUM_CORES + lax.axis_index("c")
    base = wid * BPW
    pltpu.sync_copy(user_h.at[pl.ds(base, BPW)], idx_u)
    pltpu.sync_copy(item_i_h.at[pl.ds(base, BPW)], idx_i)
    pltpu.sync_copy(item_j_h.at[pl.ds(base, BPW)], idx_j)

    lanes = lax.iota(jnp.int32, L)

    def chunk_body(c, carry):
        rbase = c * L
        iu = idx_u[pl.ds(rbase, L)]
        ii = idx_i[pl.ds(rbase, L)]
        ij = idx_j[pl.ds(rbase, L)]
        copies = []
        for t in range(L):
            gu = (iu[t] >> 3) * TG
            gi = (ii[t] >> 3) * TG
            gj = (ij[t] >> 3) * TG
            copies.append(
                pltpu.async_copy(eu_h.at[pl.ds(gu, TG)], tb_u.at[t], sem))
            copies.append(
                pltpu.async_copy(ei_h.at[pl.ds(gi, TG)], tb_i.at[t], sem))
            copies.append(
                pltpu.async_copy(ei_h.at[pl.ds(gj, TG)], tb_j.at[t], sem))
        for cp in copies:
            cp.wait()
        ru = iu & 7
        ri = ii & 7
        rj = ij & 7
        acc_i = jnp.zeros((L,), jnp.float32)
        acc_j = jnp.zeros((L,), jnp.float32)
        for k in range(D):
            col = jnp.full((L,), k, dtype=jnp.int32)
            u = plsc.load_gather(tb_u, [lanes, ru, col])
            vi = plsc.load_gather(tb_i, [lanes, ri, col])
            vj = plsc.load_gather(tb_j, [lanes, rj, col])
            acc_i = acc_i + u * vi
            acc_j = acc_j + u * vj
        oi_v[pl.ds(rbase, L)] = acc_i
        oj_v[pl.ds(rbase, L)] = acc_j
        return carry

    lax.fori_loop(0, BPW // L, chunk_body, 0)

    pltpu.sync_copy(oi_v, oi_h.at[pl.ds(base, BPW)])
    pltpu.sync_copy(oj_v, oj_h.at[pl.ds(base, BPW)])


def kernel(user, item_i, item_j, embed_user, embed_item):
    return _mfbpr(user.astype(jnp.int32), item_i.astype(jnp.int32),
                  item_j.astype(jnp.int32), embed_user, embed_item)
